# baseline (speedup 1.0000x reference)
import jax
import jax.numpy as jnp
from jax.experimental import pallas as pl
from jax.experimental.pallas import tpu as pltpu

_NCH = 8


def _pos_embed_kernel(B, F, L, emb_ref, out_ref, in_v, t_v, in_sems, out_sems):
    LC = L // _NCH

    def in_copy(c):
        return pltpu.make_async_copy(
            emb_ref.at[pl.ds(c * LC, LC), :], in_v.at[c], in_sems.at[c]
        )

    def out_copy(c, b):
        return pltpu.make_async_copy(
            t_v.at[c], out_ref.at[b, :, pl.ds(c * LC, LC)], out_sems.at[c, b]
        )

    for c in range(_NCH):
        in_copy(c).start()
    for c in range(_NCH):
        in_copy(c).wait()
        t_v[c] = in_v[c].T
        for b in range(B):
            out_copy(c, b).start()
    for c in range(_NCH):
        for b in range(B):
            out_copy(c, b).wait()


def kernel(x, mask, row_embed):
    B = x.shape[0]
    F = x.shape[1]
    L = x.shape[-1]
    LC = L // _NCH
    import functools

    return pl.pallas_call(
        functools.partial(_pos_embed_kernel, B, F, L),
        grid=(1,),
        in_specs=[pl.BlockSpec(memory_space=pl.ANY)],
        out_specs=pl.BlockSpec(memory_space=pl.ANY),
        out_shape=jax.ShapeDtypeStruct((B, F, L), jnp.float32),
        scratch_shapes=[
            pltpu.VMEM((_NCH, LC, F), jnp.float32),
            pltpu.VMEM((_NCH, F, LC), jnp.float32),
            pltpu.SemaphoreType.DMA((_NCH,)),
            pltpu.SemaphoreType.DMA((_NCH, B)),
        ],
    )(row_embed)


# speedup vs baseline: 1.0184x; 1.0184x over previous
import jax
import jax.numpy as jnp
from jax.experimental import pallas as pl
from jax.experimental.pallas import tpu as pltpu

_NCH = 2


def _pos_embed_kernel(B, F, L, emb_ref, out_ref, in_v, t_v, in_sems, out_sems):
    LC = L // _NCH

    def in_copy(c):
        return pltpu.make_async_copy(
            emb_ref.at[pl.ds(c * LC, LC), :], in_v.at[c], in_sems.at[c]
        )

    def out_copy(c, b):
        return pltpu.make_async_copy(
            t_v.at[c], out_ref.at[b, :, pl.ds(c * LC, LC)], out_sems.at[c, b]
        )

    for c in range(_NCH):
        in_copy(c).start()
    for c in range(_NCH):
        in_copy(c).wait()
        t_v[c] = in_v[c].T
        for b in range(B):
            out_copy(c, b).start()
    for c in range(_NCH):
        for b in range(B):
            out_copy(c, b).wait()


def kernel(x, mask, row_embed):
    B = x.shape[0]
    F = x.shape[1]
    L = x.shape[-1]
    LC = L // _NCH
    import functools

    return pl.pallas_call(
        functools.partial(_pos_embed_kernel, B, F, L),
        grid=(1,),
        in_specs=[pl.BlockSpec(memory_space=pl.ANY)],
        out_specs=pl.BlockSpec(memory_space=pl.ANY),
        out_shape=jax.ShapeDtypeStruct((B, F, L), jnp.float32),
        scratch_shapes=[
            pltpu.VMEM((_NCH, LC, F), jnp.float32),
            pltpu.VMEM((_NCH, F, LC), jnp.float32),
            pltpu.SemaphoreType.DMA((_NCH,)),
            pltpu.SemaphoreType.DMA((_NCH, B)),
        ],
    )(row_embed)


# speedup vs baseline: 1.0245x; 1.0060x over previous
import jax
import jax.numpy as jnp
from jax.experimental import pallas as pl
from jax.experimental.pallas import tpu as pltpu

_NCH = 4


def _pos_embed_kernel(B, F, L, emb_ref, out_ref, in_v, t_v, in_sems, out_sems):
    LC = L // _NCH

    def in_copy(c):
        return pltpu.make_async_copy(
            emb_ref.at[pl.ds(c * LC, LC), :], in_v.at[c], in_sems.at[c]
        )

    def out_copy(c, b):
        return pltpu.make_async_copy(
            t_v.at[c], out_ref.at[b, :, pl.ds(c * LC, LC)], out_sems.at[c, b]
        )

    for c in range(_NCH):
        in_copy(c).start()
    for c in range(_NCH):
        in_copy(c).wait()
        t_v[c] = in_v[c].T
        for b in range(B):
            out_copy(c, b).start()
    for c in range(_NCH):
        for b in range(B):
            out_copy(c, b).wait()


def kernel(x, mask, row_embed):
    B = x.shape[0]
    F = x.shape[1]
    L = x.shape[-1]
    LC = L // _NCH
    import functools

    return pl.pallas_call(
        functools.partial(_pos_embed_kernel, B, F, L),
        grid=(1,),
        in_specs=[pl.BlockSpec(memory_space=pl.ANY)],
        out_specs=pl.BlockSpec(memory_space=pl.ANY),
        out_shape=jax.ShapeDtypeStruct((B, F, L), jnp.float32),
        scratch_shapes=[
            pltpu.VMEM((_NCH, LC, F), jnp.float32),
            pltpu.VMEM((_NCH, F, LC), jnp.float32),
            pltpu.SemaphoreType.DMA((_NCH,)),
            pltpu.SemaphoreType.DMA((_NCH, B)),
        ],
    )(row_embed)


# speedup vs baseline: 1.0500x; 1.0249x over previous
import functools

import jax
import jax.numpy as jnp
from jax.experimental import pallas as pl
from jax.experimental.pallas import tpu as pltpu

_CHUNKS = ((0, 128), (128, 128), (256, 256), (512, 512))


def _pos_embed_kernel(B, emb_ref, out_ref, *refs):
    n = len(_CHUNKS)
    in_v = refs[:n]
    t_v = refs[n : 2 * n]
    in_sems, out_sems = refs[2 * n], refs[2 * n + 1]

    def in_copy(c):
        off, sz = _CHUNKS[c]
        return pltpu.make_async_copy(
            emb_ref.at[pl.ds(off, sz), :], in_v[c], in_sems.at[c]
        )

    def out_copy(c, b):
        off, sz = _CHUNKS[c]
        return pltpu.make_async_copy(
            t_v[c], out_ref.at[b, :, pl.ds(off, sz)], out_sems.at[c, b]
        )

    for c in range(n):
        in_copy(c).start()
    for c in range(n):
        in_copy(c).wait()
        t_v[c][...] = in_v[c][...].T
        for b in range(B):
            out_copy(c, b).start()
    for c in range(n):
        for b in range(B):
            out_copy(c, b).wait()


def kernel(x, mask, row_embed):
    B = x.shape[0]
    F = x.shape[1]
    L = x.shape[-1]
    n = len(_CHUNKS)
    return pl.pallas_call(
        functools.partial(_pos_embed_kernel, B),
        grid=(1,),
        in_specs=[pl.BlockSpec(memory_space=pl.ANY)],
        out_specs=pl.BlockSpec(memory_space=pl.ANY),
        out_shape=jax.ShapeDtypeStruct((B, F, L), jnp.float32),
        scratch_shapes=(
            [pltpu.VMEM((sz, F), jnp.float32) for _, sz in _CHUNKS]
            + [pltpu.VMEM((F, sz), jnp.float32) for _, sz in _CHUNKS]
            + [pltpu.SemaphoreType.DMA((n,)), pltpu.SemaphoreType.DMA((n, B))]
        ),
    )(row_embed)


# speedup vs baseline: 1.0525x; 1.0024x over previous
import functools

import jax
import jax.numpy as jnp
from jax.experimental import pallas as pl
from jax.experimental.pallas import tpu as pltpu

_CHUNKS = ((0, 128), (128, 384), (512, 512))


def _pos_embed_kernel(B, emb_ref, out_ref, *refs):
    n = len(_CHUNKS)
    in_v = refs[:n]
    t_v = refs[n : 2 * n]
    in_sems, out_sems = refs[2 * n], refs[2 * n + 1]

    def in_copy(c):
        off, sz = _CHUNKS[c]
        return pltpu.make_async_copy(
            emb_ref.at[pl.ds(off, sz), :], in_v[c], in_sems.at[c]
        )

    def out_copy(c, b):
        off, sz = _CHUNKS[c]
        return pltpu.make_async_copy(
            t_v[c], out_ref.at[b, :, pl.ds(off, sz)], out_sems.at[c, b]
        )

    for c in range(n):
        in_copy(c).start()
    for c in range(n):
        in_copy(c).wait()
        t_v[c][...] = in_v[c][...].T
        for b in range(B):
            out_copy(c, b).start()
    for c in range(n):
        for b in range(B):
            out_copy(c, b).wait()


def kernel(x, mask, row_embed):
    B = x.shape[0]
    F = x.shape[1]
    L = x.shape[-1]
    n = len(_CHUNKS)
    return pl.pallas_call(
        functools.partial(_pos_embed_kernel, B),
        grid=(1,),
        in_specs=[pl.BlockSpec(memory_space=pl.ANY)],
        out_specs=pl.BlockSpec(memory_space=pl.ANY),
        out_shape=jax.ShapeDtypeStruct((B, F, L), jnp.float32),
        scratch_shapes=(
            [pltpu.VMEM((sz, F), jnp.float32) for _, sz in _CHUNKS]
            + [pltpu.VMEM((F, sz), jnp.float32) for _, sz in _CHUNKS]
            + [pltpu.SemaphoreType.DMA((n,)), pltpu.SemaphoreType.DMA((n, B))]
        ),
    )(row_embed)
